# Initial kernel scaffold; baseline (speedup 1.0000x reference)
import jax
import jax.numpy as jnp
from jax.experimental import pallas as pl


def kernel(x, A, g0, b0, Wg0, bg0, edge0, sg0, sb0, Wt0, bt0, tg0, tb0, Wg1, bg1, edge1, sg1, sb1, Wt1, bt1, tg1, tb1, Wg2, bg2, edge2, sg2, sb2, Wt2, bt2, tg2, tb2, Wr2, br2, rg2, rb2):
    raise NotImplementedError("write your pallas kernel here")


# speedup vs baseline: 4.5101x; 4.5101x over previous
import jax
import jax.numpy as jnp
from jax.experimental import pallas as pl
from jax.experimental.pallas import tpu as pltpu

EPS = 1e-5
V = 25
K = 3
NTAP = 9
PAD = 4


def _stats_in_body(x_ref, g_ref, o_ref):
    x = x_ref[...]
    s1 = jnp.sum(x, axis=1, keepdims=True)
    s2 = jnp.sum(x * x, axis=1, keepdims=True)
    g = g_ref[...]
    o_ref[:, 0:1] = jnp.dot(g, s1, preferred_element_type=jnp.float32)
    o_ref[:, 1:2] = jnp.dot(g, s2, preferred_element_type=jnp.float32)


def _sgc0_body(x_ref, m_ref, b_ref, g_ref, u_ref, st_ref):
    u = jnp.dot(m_ref[...], x_ref[0], preferred_element_type=jnp.float32)
    u = u + b_ref[...]
    u_ref[0] = u
    s1 = jnp.sum(u, axis=1, keepdims=True)
    s2 = jnp.sum(u * u, axis=1, keepdims=True)
    g = g_ref[...]
    st_ref[0, :, 0:1] = jnp.dot(g, s1, preferred_element_type=jnp.float32)
    st_ref[0, :, 1:2] = jnp.dot(g, s2, preferred_element_type=jnp.float32)


def _tconv_body(u_ref, aff_ref, w_ref, bt_ref, mask_ref, v_ref, st_ref, hp_ref):
    c, vt = v_ref.shape[1], v_ref.shape[2]
    u = u_ref[0]
    h = jnp.maximum(u * aff_ref[:, 0:1] + aff_ref[:, 1:2], 0.0)
    hp_ref[:, PAD:PAD + vt] = h
    hp_ref[:, 0:PAD] = jnp.zeros((c, PAD), jnp.float32)
    hp_ref[:, PAD + vt:] = jnp.zeros((c, PAD), jnp.float32)
    acc = jnp.zeros((c, vt), jnp.float32)
    for dt in range(NTAP):
        hs = hp_ref[:, dt:dt + vt] * mask_ref[dt:dt + 1, :]
        acc = acc + jnp.dot(w_ref[:, dt * c:(dt + 1) * c], hs,
                            preferred_element_type=jnp.float32)
    vv = acc + bt_ref[...]
    v_ref[0] = vv
    st_ref[0, :, 0:1] = jnp.sum(vv, axis=1, keepdims=True)
    st_ref[0, :, 1:2] = jnp.sum(vv * vv, axis=1, keepdims=True)


def _finish_sgc_body(v_ref, aff_ref, m_ref, b_ref, g_ref, x_ref, u_ref, st_ref):
    x = jnp.maximum(v_ref[0] * aff_ref[:, 0:1] + aff_ref[:, 1:2], 0.0)
    x_ref[0] = x
    u = jnp.dot(m_ref[...], x, preferred_element_type=jnp.float32) + b_ref[...]
    u_ref[0] = u
    s1 = jnp.sum(u, axis=1, keepdims=True)
    s2 = jnp.sum(u * u, axis=1, keepdims=True)
    g = g_ref[...]
    st_ref[0, :, 0:1] = jnp.dot(g, s1, preferred_element_type=jnp.float32)
    st_ref[0, :, 1:2] = jnp.dot(g, s2, preferred_element_type=jnp.float32)


def _finish_res_sgc_body(v_ref, aff_ref, r_ref, m_ref, b_ref, g_ref,
                         x_ref, u_ref, st_ref):
    x = jnp.maximum(v_ref[0] * aff_ref[:, 0:1] + aff_ref[:, 1:2] + r_ref[0],
                    0.0)
    x_ref[0] = x
    u = jnp.dot(m_ref[...], x, preferred_element_type=jnp.float32) + b_ref[...]
    u_ref[0] = u
    s1 = jnp.sum(u, axis=1, keepdims=True)
    s2 = jnp.sum(u * u, axis=1, keepdims=True)
    g = g_ref[...]
    st_ref[0, :, 0:1] = jnp.dot(g, s1, preferred_element_type=jnp.float32)
    st_ref[0, :, 1:2] = jnp.dot(g, s2, preferred_element_type=jnp.float32)


def _proj_body(x_ref, wr_ref, br_ref, r_ref, st_ref):
    rz = jnp.dot(wr_ref[...], x_ref[0], preferred_element_type=jnp.float32)
    rz = rz + br_ref[...]
    r_ref[0] = rz
    st_ref[0, :, 0:1] = jnp.sum(rz, axis=1, keepdims=True)
    st_ref[0, :, 1:2] = jnp.sum(rz * rz, axis=1, keepdims=True)


def _final_body(v_ref, affv_ref, r_ref, affr_ref, o_ref):
    vn = v_ref[0] * affv_ref[:, 0:1] + affv_ref[:, 1:2]
    rn = r_ref[0] * affr_ref[:, 0:1] + affr_ref[:, 1:2]
    o_ref[0] = jnp.maximum(vn + rn, 0.0)


def _full(shape):
    return pl.BlockSpec(shape, lambda n: (0,) * len(shape))


def _pern(shape):
    return pl.BlockSpec((1,) + shape, lambda n: (n, 0, 0))


def _finalize(st, gamma, beta, count):
    tot = jnp.sum(st, axis=0)
    mean = tot[:, 0] / count
    var = tot[:, 1] / count - mean * mean
    scale = gamma / jnp.sqrt(var + EPS)
    shift = beta - mean * scale
    return jnp.stack([scale, shift], axis=1)


def _build_m(Wg, bg, A, edge, c_in, c_out):
    b = A * edge
    wr = Wg.reshape(K, c_out, c_in)
    m = jnp.einsum('kci,kvw->cwiv', wr, b).reshape(c_out * V, c_in * V)
    bias = jnp.einsum('kc,kw->cw', bg.reshape(K, c_out),
                      jnp.sum(b, axis=1)).reshape(c_out * V, 1)
    return m, bias


def _wstack(Wt):
    o, i, taps, _ = Wt.shape
    return Wt[:, :, :, 0].transpose(0, 2, 1).reshape(o, taps * i)


def _group(c, rep):
    return jnp.repeat(jnp.eye(c, dtype=jnp.float32), rep, axis=1)


def kernel(x, A, g0, b0, Wg0, bg0, edge0, sg0, sb0, Wt0, bt0, tg0, tb0,
           Wg1, bg1, edge1, sg1, sb1, Wt1, bt1, tg1, tb1,
           Wg2, bg2, edge2, sg2, sb2, Wt2, bt2, tg2, tb2, Wr2, br2, rg2, rb2):
    n, c0, t, v = x.shape
    assert v == V
    c1 = sg0.shape[0]
    c2 = sg1.shape[0]
    c3 = sg2.shape[0]
    vt = v * t
    cnt = jnp.float32(n * t * v)
    f32 = jnp.float32

    x2d = x.reshape(n * c0, t * v)
    g_in = jnp.tile(jnp.eye(c0, dtype=f32), (1, n))
    st_in = pl.pallas_call(
        _stats_in_body,
        out_shape=jax.ShapeDtypeStruct((c0, 2), f32),
        in_specs=[pl.BlockSpec((n * c0, t * v), lambda: (0, 0)),
                  pl.BlockSpec((c0, n * c0), lambda: (0, 0))],
        out_specs=pl.BlockSpec((c0, 2), lambda: (0, 0)),
    )(x2d, g_in)
    mean0 = st_in[:, 0] / cnt
    var0 = st_in[:, 1] / cnt - mean0 * mean0
    sc0 = g0 / jnp.sqrt(var0 + EPS)
    sh0 = b0 - mean0 * sc0

    m0, bias0 = _build_m(Wg0, bg0, A, edge0, c0, c1)
    m0f = m0 * jnp.repeat(sc0, V)[None, :]
    bias0f = bias0 + m0 @ jnp.repeat(sh0, V)[:, None]

    m1, bias1 = _build_m(Wg1, bg1, A, edge1, c1, c2)
    m2, bias2 = _build_m(Wg2, bg2, A, edge2, c2, c3)
    wt0 = _wstack(Wt0)
    wt1 = _wstack(Wt1)
    wt2 = _wstack(Wt2)
    g1 = _group(c1, V)
    g2 = _group(c2, V)
    g3 = _group(c3, V)

    tmod = jnp.arange(vt, dtype=jnp.int32) % t
    off = jnp.arange(NTAP, dtype=jnp.int32)[:, None] - PAD
    masks = ((tmod[None, :] + off >= 0) & (tmod[None, :] + off < t)).astype(f32)

    xt = x.transpose(0, 1, 3, 2).reshape(n, c0 * V, t)

    u0, st_u0 = pl.pallas_call(
        _sgc0_body,
        grid=(n,),
        out_shape=(jax.ShapeDtypeStruct((n, c1 * V, t), f32),
                   jax.ShapeDtypeStruct((n, c1, 2), f32)),
        in_specs=[_pern((c0 * V, t)), _full((c1 * V, c0 * V)),
                  _full((c1 * V, 1)), _full((c1, c1 * V))],
        out_specs=(_pern((c1 * V, t)), _pern((c1, 2))),
    )(xt, m0f, bias0f, g1)
    aff_u0 = _finalize(st_u0, sg0, sb0, cnt)

    v0, st_v0 = pl.pallas_call(
        _tconv_body,
        grid=(n,),
        out_shape=(jax.ShapeDtypeStruct((n, c1, vt), f32),
                   jax.ShapeDtypeStruct((n, c1, 2), f32)),
        in_specs=[_pern((c1, vt)), _full((c1, 2)),
                  _full((c1, NTAP * c1)), _full((c1, 1)), _full((NTAP, vt))],
        out_specs=(_pern((c1, vt)), _pern((c1, 2))),
        scratch_shapes=[pltpu.VMEM((c1, vt + 2 * PAD), f32)],
    )(u0.reshape(n, c1, vt), aff_u0, wt0, bt0.reshape(c1, 1), masks)
    aff_v0 = jnp.repeat(_finalize(st_v0, tg0, tb0, cnt), V, axis=0)

    x1, u1, st_u1 = pl.pallas_call(
        _finish_sgc_body,
        grid=(n,),
        out_shape=(jax.ShapeDtypeStruct((n, c1 * V, t), f32),
                   jax.ShapeDtypeStruct((n, c2 * V, t), f32),
                   jax.ShapeDtypeStruct((n, c2, 2), f32)),
        in_specs=[_pern((c1 * V, t)), _full((c1 * V, 2)),
                  _full((c2 * V, c1 * V)), _full((c2 * V, 1)),
                  _full((c2, c2 * V))],
        out_specs=(_pern((c1 * V, t)), _pern((c2 * V, t)), _pern((c2, 2))),
    )(v0.reshape(n, c1 * V, t), aff_v0, m1, bias1, g2)
    aff_u1 = _finalize(st_u1, sg1, sb1, cnt)

    v1, st_v1 = pl.pallas_call(
        _tconv_body,
        grid=(n,),
        out_shape=(jax.ShapeDtypeStruct((n, c2, vt), f32),
                   jax.ShapeDtypeStruct((n, c2, 2), f32)),
        in_specs=[_pern((c2, vt)), _full((c2, 2)),
                  _full((c2, NTAP * c2)), _full((c2, 1)), _full((NTAP, vt))],
        out_specs=(_pern((c2, vt)), _pern((c2, 2))),
        scratch_shapes=[pltpu.VMEM((c2, vt + 2 * PAD), f32)],
    )(u1.reshape(n, c2, vt), aff_u1, wt1, bt1.reshape(c2, 1), masks)
    aff_v1 = jnp.repeat(_finalize(st_v1, tg1, tb1, cnt), V, axis=0)

    x2, u2, st_u2 = pl.pallas_call(
        _finish_res_sgc_body,
        grid=(n,),
        out_shape=(jax.ShapeDtypeStruct((n, c2 * V, t), f32),
                   jax.ShapeDtypeStruct((n, c3 * V, t), f32),
                   jax.ShapeDtypeStruct((n, c3, 2), f32)),
        in_specs=[_pern((c2 * V, t)), _full((c2 * V, 2)), _pern((c2 * V, t)),
                  _full((c3 * V, c2 * V)), _full((c3 * V, 1)),
                  _full((c3, c3 * V))],
        out_specs=(_pern((c2 * V, t)), _pern((c3 * V, t)), _pern((c3, 2))),
    )(v1.reshape(n, c2 * V, t), aff_v1, x1, m2, bias2, g3)
    aff_u2 = _finalize(st_u2, sg2, sb2, cnt)

    r2, st_r2 = pl.pallas_call(
        _proj_body,
        grid=(n,),
        out_shape=(jax.ShapeDtypeStruct((n, c3, vt), f32),
                   jax.ShapeDtypeStruct((n, c3, 2), f32)),
        in_specs=[_pern((c2, vt)), _full((c3, c2)), _full((c3, 1))],
        out_specs=(_pern((c3, vt)), _pern((c3, 2))),
    )(x2.reshape(n, c2, vt), Wr2, br2.reshape(c3, 1))
    aff_r2 = _finalize(st_r2, rg2, rb2, cnt)

    v2, st_v2 = pl.pallas_call(
        _tconv_body,
        grid=(n,),
        out_shape=(jax.ShapeDtypeStruct((n, c3, vt), f32),
                   jax.ShapeDtypeStruct((n, c3, 2), f32)),
        in_specs=[_pern((c3, vt)), _full((c3, 2)),
                  _full((c3, NTAP * c3)), _full((c3, 1)), _full((NTAP, vt))],
        out_specs=(_pern((c3, vt)), _pern((c3, 2))),
        scratch_shapes=[pltpu.VMEM((c3, vt + 2 * PAD), f32)],
    )(u2.reshape(n, c3, vt), aff_u2, wt2, bt2.reshape(c3, 1), masks)
    aff_v2 = _finalize(st_v2, tg2, tb2, cnt)

    out = pl.pallas_call(
        _final_body,
        grid=(n,),
        out_shape=jax.ShapeDtypeStruct((n, c3, vt), f32),
        in_specs=[_pern((c3, vt)), _full((c3, 2)),
                  _pern((c3, vt)), _full((c3, 2))],
        out_specs=_pern((c3, vt)),
    )(v2, aff_v2, r2, aff_r2)

    return out.reshape(n, c3, v, t).transpose(0, 1, 3, 2)


# speedup vs baseline: 6.4472x; 1.4295x over previous
import functools

import jax
import jax.numpy as jnp
from jax.experimental import pallas as pl
from jax.experimental.pallas import tpu as pltpu

EPS = 1e-5
V = 25
K = 3
NTAP = 9
PAD = 4


def _finalize(st_ref, g_ref, be_ref, cnt):
    tot = jnp.sum(st_ref[...], axis=0)
    mean = tot[:, 0:1] / cnt
    var = tot[:, 1:2] / cnt - mean * mean
    scale = g_ref[...] * jax.lax.rsqrt(var + EPS)
    shift = be_ref[...] - mean * scale
    return scale, shift


def _stats_in_body(x_ref, g_ref, o_ref):
    x = x_ref[...]
    s1 = jnp.sum(x, axis=1, keepdims=True)
    s2 = jnp.sum(x * x, axis=1, keepdims=True)
    g = g_ref[...]
    o_ref[:, 0:1] = jnp.dot(g, s1, preferred_element_type=jnp.float32)
    o_ref[:, 1:2] = jnp.dot(g, s2, preferred_element_type=jnp.float32)


def _sgc0_body(x_ref, st_ref, g0_ref, b0_ref, m_ref, b_ref, gm_ref,
               u_ref, sto_ref, *, nbatch):
    t = x_ref.shape[2]
    cnt = jnp.float32(nbatch * t * V)
    tot = st_ref[...]
    mean = tot[:, 0:1] / cnt
    var = tot[:, 1:2] / cnt - mean * mean
    scale = g0_ref[...] * jax.lax.rsqrt(var + EPS)
    shift = b0_ref[...] - mean * scale
    s_rows = jnp.concatenate([scale] * V, axis=0)
    t_rows = jnp.concatenate([shift] * V, axis=0)
    xn = x_ref[0] * s_rows + t_rows
    u = jnp.dot(m_ref[...], xn, preferred_element_type=jnp.float32)
    u = u + b_ref[...]
    u_ref[0] = u
    s1 = jnp.sum(u, axis=1, keepdims=True)
    s2 = jnp.sum(u * u, axis=1, keepdims=True)
    g = gm_ref[...]
    sto_ref[0, :, 0:1] = jnp.dot(g, s1, preferred_element_type=jnp.float32)
    sto_ref[0, :, 1:2] = jnp.dot(g, s2, preferred_element_type=jnp.float32)


def _tconv_body(u_ref, st_ref, g_ref, be_ref, w_ref, bt_ref, mask_ref,
                v_ref, sto_ref, hp_ref):
    c, vt = v_ref.shape[1], v_ref.shape[2]
    t = vt // V
    cnt = jnp.float32(st_ref.shape[0] * t * V)
    scale, shift = _finalize(st_ref, g_ref, be_ref, cnt)
    for vv_ in range(V):
        blk = u_ref[0, vv_ * c:(vv_ + 1) * c, :]
        h = jnp.maximum(blk * scale + shift, 0.0)
        hp_ref[:, PAD + vv_ * t:PAD + (vv_ + 1) * t] = h
    hp_ref[:, 0:PAD] = jnp.zeros((c, PAD), jnp.float32)
    hp_ref[:, PAD + vt:] = jnp.zeros((c, PAD), jnp.float32)
    acc = jnp.zeros((c, vt), jnp.float32)
    for dt in range(NTAP):
        hs = hp_ref[:, dt:dt + vt] * mask_ref[dt:dt + 1, :]
        acc = acc + jnp.dot(w_ref[:, dt * c:(dt + 1) * c], hs,
                            preferred_element_type=jnp.float32)
    out = acc + bt_ref[...]
    v_ref[0] = out
    sto_ref[0, :, 0:1] = jnp.sum(out, axis=1, keepdims=True)
    sto_ref[0, :, 1:2] = jnp.sum(out * out, axis=1, keepdims=True)


def _finish_sgc_body(v_ref, st_ref, g_ref, be_ref, m_ref, b_ref, gm_ref,
                     x_ref, u_ref, sto_ref, xs_ref):
    c, vt = v_ref.shape[1], v_ref.shape[2]
    t = vt // V
    cnt = jnp.float32(st_ref.shape[0] * t * V)
    scale, shift = _finalize(st_ref, g_ref, be_ref, cnt)
    x = jnp.maximum(v_ref[0] * scale + shift, 0.0)
    x_ref[0] = x
    for vv_ in range(V):
        xs_ref[vv_ * c:(vv_ + 1) * c, :] = x[:, vv_ * t:(vv_ + 1) * t]
    u = jnp.dot(m_ref[...], xs_ref[...],
                preferred_element_type=jnp.float32) + b_ref[...]
    u_ref[0] = u
    s1 = jnp.sum(u, axis=1, keepdims=True)
    s2 = jnp.sum(u * u, axis=1, keepdims=True)
    g = gm_ref[...]
    sto_ref[0, :, 0:1] = jnp.dot(g, s1, preferred_element_type=jnp.float32)
    sto_ref[0, :, 1:2] = jnp.dot(g, s2, preferred_element_type=jnp.float32)


def _finish_res_sgc_body(v_ref, st_ref, g_ref, be_ref, r_ref, m_ref, b_ref,
                         gm_ref, x_ref, u_ref, sto_ref, xs_ref):
    c, vt = v_ref.shape[1], v_ref.shape[2]
    t = vt // V
    cnt = jnp.float32(st_ref.shape[0] * t * V)
    scale, shift = _finalize(st_ref, g_ref, be_ref, cnt)
    x = jnp.maximum(v_ref[0] * scale + shift + r_ref[0], 0.0)
    x_ref[0] = x
    for vv_ in range(V):
        xs_ref[vv_ * c:(vv_ + 1) * c, :] = x[:, vv_ * t:(vv_ + 1) * t]
    u = jnp.dot(m_ref[...], xs_ref[...],
                preferred_element_type=jnp.float32) + b_ref[...]
    u_ref[0] = u
    s1 = jnp.sum(u, axis=1, keepdims=True)
    s2 = jnp.sum(u * u, axis=1, keepdims=True)
    g = gm_ref[...]
    sto_ref[0, :, 0:1] = jnp.dot(g, s1, preferred_element_type=jnp.float32)
    sto_ref[0, :, 1:2] = jnp.dot(g, s2, preferred_element_type=jnp.float32)


def _proj_body(x_ref, wr_ref, br_ref, r_ref, sto_ref):
    rz = jnp.dot(wr_ref[...], x_ref[0], preferred_element_type=jnp.float32)
    rz = rz + br_ref[...]
    r_ref[0] = rz
    sto_ref[0, :, 0:1] = jnp.sum(rz, axis=1, keepdims=True)
    sto_ref[0, :, 1:2] = jnp.sum(rz * rz, axis=1, keepdims=True)


def _final_body(v_ref, stv_ref, gv_ref, bev_ref, r_ref, str_ref, gr_ref,
                ber_ref, o_ref):
    vt = v_ref.shape[2]
    t = vt // V
    cnt = jnp.float32(stv_ref.shape[0] * t * V)
    sv, bv = _finalize(stv_ref, gv_ref, bev_ref, cnt)
    sr, br = _finalize(str_ref, gr_ref, ber_ref, cnt)
    o_ref[0] = jnp.maximum(v_ref[0] * sv + bv + r_ref[0] * sr + br, 0.0)


def _full(shape):
    return pl.BlockSpec(shape, lambda n: (0,) * len(shape))


def _pern(shape):
    return pl.BlockSpec((1,) + shape, lambda n: (n, 0, 0))


def _build_m(Wg, bg, A, edge, c_in, c_out):
    b = A * edge
    wr = Wg.reshape(K, c_out, c_in)
    m = jnp.einsum('kci,kvw->wcvi', wr, b).reshape(c_out * V, c_in * V)
    bias = jnp.einsum('kc,kw->wc', bg.reshape(K, c_out),
                      jnp.sum(b, axis=1)).reshape(c_out * V, 1)
    return m, bias


def _wstack(Wt):
    o, i, taps, _ = Wt.shape
    return Wt[:, :, :, 0].transpose(0, 2, 1).reshape(o, taps * i)


def _group(c):
    return jnp.tile(jnp.eye(c, dtype=jnp.float32), (1, V))


def kernel(x, A, g0, b0, Wg0, bg0, edge0, sg0, sb0, Wt0, bt0, tg0, tb0,
           Wg1, bg1, edge1, sg1, sb1, Wt1, bt1, tg1, tb1,
           Wg2, bg2, edge2, sg2, sb2, Wt2, bt2, tg2, tb2, Wr2, br2, rg2, rb2):
    n, c0, t, v = x.shape
    assert v == V
    c1 = sg0.shape[0]
    c2 = sg1.shape[0]
    c3 = sg2.shape[0]
    vt = v * t
    f32 = jnp.float32
    col = lambda a: a.reshape(-1, 1)

    m0, bias0 = _build_m(Wg0, bg0, A, edge0, c0, c1)
    m1, bias1 = _build_m(Wg1, bg1, A, edge1, c1, c2)
    m2, bias2 = _build_m(Wg2, bg2, A, edge2, c2, c3)
    wt0 = _wstack(Wt0)
    wt1 = _wstack(Wt1)
    wt2 = _wstack(Wt2)
    g1 = _group(c1)
    g2 = _group(c2)
    g3 = _group(c3)

    tmod = jnp.arange(vt, dtype=jnp.int32) % t
    off = jnp.arange(NTAP, dtype=jnp.int32)[:, None] - PAD
    masks = ((tmod[None, :] + off >= 0) & (tmod[None, :] + off < t)).astype(f32)

    x2d = x.reshape(n * c0, t * v)
    g_in = jnp.tile(jnp.eye(c0, dtype=f32), (1, n))
    st_in = pl.pallas_call(
        _stats_in_body,
        out_shape=jax.ShapeDtypeStruct((c0, 2), f32),
        in_specs=[pl.BlockSpec((n * c0, t * v), lambda: (0, 0)),
                  pl.BlockSpec((c0, n * c0), lambda: (0, 0))],
        out_specs=pl.BlockSpec((c0, 2), lambda: (0, 0)),
    )(x2d, g_in)

    xt = x.transpose(0, 3, 1, 2).reshape(n, v * c0, t)

    u0, st_u0 = pl.pallas_call(
        functools.partial(_sgc0_body, nbatch=n),
        grid=(n,),
        out_shape=(jax.ShapeDtypeStruct((n, v * c1, t), f32),
                   jax.ShapeDtypeStruct((n, c1, 2), f32)),
        in_specs=[_pern((v * c0, t)), _full((c0, 2)), _full((c0, 1)),
                  _full((c0, 1)), _full((v * c1, v * c0)), _full((v * c1, 1)),
                  _full((c1, v * c1))],
        out_specs=(_pern((v * c1, t)), _pern((c1, 2))),
    )(xt, st_in, col(g0), col(b0), m0, bias0, g1)

    v0, st_v0 = pl.pallas_call(
        _tconv_body,
        grid=(n,),
        out_shape=(jax.ShapeDtypeStruct((n, c1, vt), f32),
                   jax.ShapeDtypeStruct((n, c1, 2), f32)),
        in_specs=[_pern((v * c1, t)), _full((n, c1, 2)), _full((c1, 1)),
                  _full((c1, 1)), _full((c1, NTAP * c1)), _full((c1, 1)),
                  _full((NTAP, vt))],
        out_specs=(_pern((c1, vt)), _pern((c1, 2))),
        scratch_shapes=[pltpu.VMEM((c1, vt + 2 * PAD), f32)],
    )(u0, st_u0, col(sg0), col(sb0), wt0, col(bt0), masks)

    x1, u1, st_u1 = pl.pallas_call(
        _finish_sgc_body,
        grid=(n,),
        out_shape=(jax.ShapeDtypeStruct((n, c1, vt), f32),
                   jax.ShapeDtypeStruct((n, v * c2, t), f32),
                   jax.ShapeDtypeStruct((n, c2, 2), f32)),
        in_specs=[_pern((c1, vt)), _full((n, c1, 2)), _full((c1, 1)),
                  _full((c1, 1)), _full((v * c2, v * c1)), _full((v * c2, 1)),
                  _full((c2, v * c2))],
        out_specs=(_pern((c1, vt)), _pern((v * c2, t)), _pern((c2, 2))),
        scratch_shapes=[pltpu.VMEM((v * c1, t), f32)],
    )(v0, st_v0, col(tg0), col(tb0), m1, bias1, g2)

    v1, st_v1 = pl.pallas_call(
        _tconv_body,
        grid=(n,),
        out_shape=(jax.ShapeDtypeStruct((n, c2, vt), f32),
                   jax.ShapeDtypeStruct((n, c2, 2), f32)),
        in_specs=[_pern((v * c2, t)), _full((n, c2, 2)), _full((c2, 1)),
                  _full((c2, 1)), _full((c2, NTAP * c2)), _full((c2, 1)),
                  _full((NTAP, vt))],
        out_specs=(_pern((c2, vt)), _pern((c2, 2))),
        scratch_shapes=[pltpu.VMEM((c2, vt + 2 * PAD), f32)],
    )(u1, st_u1, col(sg1), col(sb1), wt1, col(bt1), masks)

    x2, u2, st_u2 = pl.pallas_call(
        _finish_res_sgc_body,
        grid=(n,),
        out_shape=(jax.ShapeDtypeStruct((n, c2, vt), f32),
                   jax.ShapeDtypeStruct((n, v * c3, t), f32),
                   jax.ShapeDtypeStruct((n, c3, 2), f32)),
        in_specs=[_pern((c2, vt)), _full((n, c2, 2)), _full((c2, 1)),
                  _full((c2, 1)), _pern((c2, vt)),
                  _full((v * c3, v * c2)), _full((v * c3, 1)),
                  _full((c3, v * c3))],
        out_specs=(_pern((c2, vt)), _pern((v * c3, t)), _pern((c3, 2))),
        scratch_shapes=[pltpu.VMEM((v * c2, t), f32)],
    )(v1, st_v1, col(tg1), col(tb1), x1, m2, bias2, g3)

    r2, st_r2 = pl.pallas_call(
        _proj_body,
        grid=(n,),
        out_shape=(jax.ShapeDtypeStruct((n, c3, vt), f32),
                   jax.ShapeDtypeStruct((n, c3, 2), f32)),
        in_specs=[_pern((c2, vt)), _full((c3, c2)), _full((c3, 1))],
        out_specs=(_pern((c3, vt)), _pern((c3, 2))),
    )(x2, Wr2, col(br2))

    v2, st_v2 = pl.pallas_call(
        _tconv_body,
        grid=(n,),
        out_shape=(jax.ShapeDtypeStruct((n, c3, vt), f32),
                   jax.ShapeDtypeStruct((n, c3, 2), f32)),
        in_specs=[_pern((v * c3, t)), _full((n, c3, 2)), _full((c3, 1)),
                  _full((c3, 1)), _full((c3, NTAP * c3)), _full((c3, 1)),
                  _full((NTAP, vt))],
        out_specs=(_pern((c3, vt)), _pern((c3, 2))),
        scratch_shapes=[pltpu.VMEM((c3, vt + 2 * PAD), f32)],
    )(u2, st_u2, col(sg2), col(sb2), wt2, col(bt2), masks)

    out = pl.pallas_call(
        _final_body,
        grid=(n,),
        out_shape=jax.ShapeDtypeStruct((n, c3, vt), f32),
        in_specs=[_pern((c3, vt)), _full((n, c3, 2)), _full((c3, 1)),
                  _full((c3, 1)), _pern((c3, vt)), _full((n, c3, 2)),
                  _full((c3, 1)), _full((c3, 1))],
        out_specs=_pern((c3, vt)),
    )(v2, st_v2, col(tg2), col(tb2), r2, st_r2, col(rg2), col(rb2))

    return out.reshape(n, c3, v, t).transpose(0, 1, 3, 2)


# speedup vs baseline: 6.8166x; 1.0573x over previous
import functools

import jax
import jax.numpy as jnp
from jax.experimental import pallas as pl
from jax.experimental.pallas import tpu as pltpu

EPS = 1e-5
V = 25
K = 3
NTAP = 9
PAD = 4


def _finalize(st_ref, g_ref, be_ref, cnt):
    tot = jnp.sum(st_ref[...], axis=0)
    mean = tot[:, 0:1] / cnt
    var = tot[:, 1:2] / cnt - mean * mean
    scale = g_ref[...] * jax.lax.rsqrt(var + EPS)
    shift = be_ref[...] - mean * scale
    return scale, shift


def _stats_in_body(x_ref, g_ref, o_ref):
    x = x_ref[...]
    s1 = jnp.sum(x, axis=1, keepdims=True)
    s2 = jnp.sum(x * x, axis=1, keepdims=True)
    g = g_ref[...]
    o_ref[:, 0:1] = jnp.dot(g, s1, preferred_element_type=jnp.float32)
    o_ref[:, 1:2] = jnp.dot(g, s2, preferred_element_type=jnp.float32)


def _sgc0_body(x_ref, st_ref, g0_ref, b0_ref, m_ref, b_ref, gm_ref,
               u_ref, sto_ref, *, nbatch):
    t = x_ref.shape[2]
    cnt = jnp.float32(nbatch * t * V)
    tot = st_ref[...]
    mean = tot[:, 0:1] / cnt
    var = tot[:, 1:2] / cnt - mean * mean
    scale = g0_ref[...] * jax.lax.rsqrt(var + EPS)
    shift = b0_ref[...] - mean * scale
    s_rows = jnp.concatenate([scale] * V, axis=0)
    t_rows = jnp.concatenate([shift] * V, axis=0)
    xn = x_ref[0] * s_rows + t_rows
    u = jnp.dot(m_ref[...], xn, preferred_element_type=jnp.float32)
    u = u + b_ref[...]
    u_ref[0] = u
    s1 = jnp.sum(u, axis=1, keepdims=True)
    s2 = jnp.sum(u * u, axis=1, keepdims=True)
    g = gm_ref[...]
    sto_ref[0, :, 0:1] = jnp.dot(g, s1, preferred_element_type=jnp.float32)
    sto_ref[0, :, 1:2] = jnp.dot(g, s2, preferred_element_type=jnp.float32)


def _tconv_core(u_ref, scale, shift, w_ref, bt_ref, gap_ref, hp_ref):
    c = w_ref.shape[0]
    t = u_ref.shape[2]
    tp = t + 2 * PAD
    vtp = V * tp
    hp_ref[...] = jnp.zeros((c, vtp + 2 * PAD), jnp.float32)
    for vv_ in range(V):
        blk = u_ref[0, vv_ * c:(vv_ + 1) * c, :]
        h = jnp.maximum(blk * scale + shift, 0.0)
        hp_ref[:, 2 * PAD + vv_ * tp:2 * PAD + vv_ * tp + t] = h
    acc = jnp.zeros((c, vtp), jnp.float32)
    for dt in range(NTAP):
        acc = acc + jnp.dot(w_ref[:, dt * c:(dt + 1) * c],
                            hp_ref[:, dt:dt + vtp],
                            preferred_element_type=jnp.float32)
    out = acc + bt_ref[...]
    return out, out * gap_ref[...]


def _tconv_body(u_ref, st_ref, g_ref, be_ref, w_ref, bt_ref, gap_ref,
                v_ref, sto_ref, hp_ref):
    t = u_ref.shape[2]
    cnt = jnp.float32(st_ref.shape[0] * t * V)
    scale, shift = _finalize(st_ref, g_ref, be_ref, cnt)
    out, outm = _tconv_core(u_ref, scale, shift, w_ref, bt_ref, gap_ref, hp_ref)
    v_ref[0] = out
    sto_ref[0, :, 0:1] = jnp.sum(outm, axis=1, keepdims=True)
    sto_ref[0, :, 1:2] = jnp.sum(outm * outm, axis=1, keepdims=True)


def _tconv_proj_body(u_ref, st_ref, g_ref, be_ref, w_ref, bt_ref, gap_ref,
                     x_ref, wr_ref, br_ref,
                     v_ref, sto_ref, r_ref, str_ref, hp_ref):
    t = u_ref.shape[2]
    cnt = jnp.float32(st_ref.shape[0] * t * V)
    scale, shift = _finalize(st_ref, g_ref, be_ref, cnt)
    out, outm = _tconv_core(u_ref, scale, shift, w_ref, bt_ref, gap_ref, hp_ref)
    v_ref[0] = out
    sto_ref[0, :, 0:1] = jnp.sum(outm, axis=1, keepdims=True)
    sto_ref[0, :, 1:2] = jnp.sum(outm * outm, axis=1, keepdims=True)
    rz = jnp.dot(wr_ref[...], x_ref[0], preferred_element_type=jnp.float32)
    rz = rz + br_ref[...]
    r_ref[0] = rz
    rzm = rz * gap_ref[...]
    str_ref[0, :, 0:1] = jnp.sum(rzm, axis=1, keepdims=True)
    str_ref[0, :, 1:2] = jnp.sum(rzm * rzm, axis=1, keepdims=True)


def _finish_sgc(v_val, r_val, st_ref, g_ref, be_ref, m_ref, b_ref, gm_ref,
                x_ref, u_ref, sto_ref, xs_ref):
    c = x_ref.shape[1]
    tp = x_ref.shape[2] // V
    t = tp - 2 * PAD
    cnt = jnp.float32(st_ref.shape[0] * t * V)
    scale, shift = _finalize(st_ref, g_ref, be_ref, cnt)
    pre = v_val * scale + shift
    if r_val is not None:
        pre = pre + r_val
    x = jnp.maximum(pre, 0.0)
    x_ref[0] = x
    for vv_ in range(V):
        xs_ref[vv_ * c:(vv_ + 1) * c, :] = x[:, vv_ * tp + PAD:vv_ * tp + PAD + t]
    u = jnp.dot(m_ref[...], xs_ref[...],
                preferred_element_type=jnp.float32) + b_ref[...]
    u_ref[0] = u
    s1 = jnp.sum(u, axis=1, keepdims=True)
    s2 = jnp.sum(u * u, axis=1, keepdims=True)
    g = gm_ref[...]
    sto_ref[0, :, 0:1] = jnp.dot(g, s1, preferred_element_type=jnp.float32)
    sto_ref[0, :, 1:2] = jnp.dot(g, s2, preferred_element_type=jnp.float32)


def _finish_sgc_body(v_ref, st_ref, g_ref, be_ref, m_ref, b_ref, gm_ref,
                     x_ref, u_ref, sto_ref, xs_ref):
    _finish_sgc(v_ref[0], None, st_ref, g_ref, be_ref, m_ref, b_ref, gm_ref,
                x_ref, u_ref, sto_ref, xs_ref)


def _finish_res_sgc_body(v_ref, st_ref, g_ref, be_ref, r_ref, m_ref, b_ref,
                         gm_ref, x_ref, u_ref, sto_ref, xs_ref):
    _finish_sgc(v_ref[0], r_ref[0], st_ref, g_ref, be_ref, m_ref, b_ref,
                gm_ref, x_ref, u_ref, sto_ref, xs_ref)


def _final_body(v_ref, stv_ref, gv_ref, bev_ref, r_ref, str_ref, gr_ref,
                ber_ref, o_ref):
    tp = v_ref.shape[2] // V
    t = tp - 2 * PAD
    cnt = jnp.float32(stv_ref.shape[0] * t * V)
    sv, bv = _finalize(stv_ref, gv_ref, bev_ref, cnt)
    sr, br = _finalize(str_ref, gr_ref, ber_ref, cnt)
    o_ref[0] = jnp.maximum(v_ref[0] * sv + bv + r_ref[0] * sr + br, 0.0)


def _full(shape):
    return pl.BlockSpec(shape, lambda n: (0,) * len(shape))


def _pern(shape):
    return pl.BlockSpec((1,) + shape, lambda n: (n, 0, 0))


def _build_m(Wg, bg, A, edge, c_in, c_out):
    b = A * edge
    wr = Wg.reshape(K, c_out, c_in)
    m = jnp.einsum('kci,kvw->wcvi', wr, b).reshape(c_out * V, c_in * V)
    bias = jnp.einsum('kc,kw->wc', bg.reshape(K, c_out),
                      jnp.sum(b, axis=1)).reshape(c_out * V, 1)
    return m, bias


def _wstack(Wt):
    o, i, taps, _ = Wt.shape
    return Wt[:, :, :, 0].transpose(0, 2, 1).reshape(o, taps * i)


def _group(c):
    return jnp.tile(jnp.eye(c, dtype=jnp.float32), (1, V))


def kernel(x, A, g0, b0, Wg0, bg0, edge0, sg0, sb0, Wt0, bt0, tg0, tb0,
           Wg1, bg1, edge1, sg1, sb1, Wt1, bt1, tg1, tb1,
           Wg2, bg2, edge2, sg2, sb2, Wt2, bt2, tg2, tb2, Wr2, br2, rg2, rb2):
    n, c0, t, v = x.shape
    assert v == V
    c1 = sg0.shape[0]
    c2 = sg1.shape[0]
    c3 = sg2.shape[0]
    tp = t + 2 * PAD
    vtp = v * tp
    f32 = jnp.float32
    col = lambda a: a.reshape(-1, 1)

    m0, bias0 = _build_m(Wg0, bg0, A, edge0, c0, c1)
    m1, bias1 = _build_m(Wg1, bg1, A, edge1, c1, c2)
    m2, bias2 = _build_m(Wg2, bg2, A, edge2, c2, c3)
    wt0 = _wstack(Wt0)
    wt1 = _wstack(Wt1)
    wt2 = _wstack(Wt2)
    g1 = _group(c1)
    g2 = _group(c2)
    g3 = _group(c3)

    tcol = jnp.arange(vtp, dtype=jnp.int32) % tp
    gap = ((tcol >= PAD) & (tcol < PAD + t)).astype(f32)[None, :]

    x2d = x.reshape(n * c0, t * v)
    g_in = jnp.tile(jnp.eye(c0, dtype=f32), (1, n))
    st_in = pl.pallas_call(
        _stats_in_body,
        out_shape=jax.ShapeDtypeStruct((c0, 2), f32),
        in_specs=[pl.BlockSpec((n * c0, t * v), lambda: (0, 0)),
                  pl.BlockSpec((c0, n * c0), lambda: (0, 0))],
        out_specs=pl.BlockSpec((c0, 2), lambda: (0, 0)),
    )(x2d, g_in)

    xt = x.transpose(0, 3, 1, 2).reshape(n, v * c0, t)

    u0, st_u0 = pl.pallas_call(
        functools.partial(_sgc0_body, nbatch=n),
        grid=(n,),
        out_shape=(jax.ShapeDtypeStruct((n, v * c1, t), f32),
                   jax.ShapeDtypeStruct((n, c1, 2), f32)),
        in_specs=[_pern((v * c0, t)), _full((c0, 2)), _full((c0, 1)),
                  _full((c0, 1)), _full((v * c1, v * c0)), _full((v * c1, 1)),
                  _full((c1, v * c1))],
        out_specs=(_pern((v * c1, t)), _pern((c1, 2))),
    )(xt, st_in, col(g0), col(b0), m0, bias0, g1)

    v0, st_v0 = pl.pallas_call(
        _tconv_body,
        grid=(n,),
        out_shape=(jax.ShapeDtypeStruct((n, c1, vtp), f32),
                   jax.ShapeDtypeStruct((n, c1, 2), f32)),
        in_specs=[_pern((v * c1, t)), _full((n, c1, 2)), _full((c1, 1)),
                  _full((c1, 1)), _full((c1, NTAP * c1)), _full((c1, 1)),
                  _full((1, vtp))],
        out_specs=(_pern((c1, vtp)), _pern((c1, 2))),
        scratch_shapes=[pltpu.VMEM((c1, vtp + 2 * PAD), f32)],
    )(u0, st_u0, col(sg0), col(sb0), wt0, col(bt0), gap)

    x1, u1, st_u1 = pl.pallas_call(
        _finish_sgc_body,
        grid=(n,),
        out_shape=(jax.ShapeDtypeStruct((n, c1, vtp), f32),
                   jax.ShapeDtypeStruct((n, v * c2, t), f32),
                   jax.ShapeDtypeStruct((n, c2, 2), f32)),
        in_specs=[_pern((c1, vtp)), _full((n, c1, 2)), _full((c1, 1)),
                  _full((c1, 1)), _full((v * c2, v * c1)), _full((v * c2, 1)),
                  _full((c2, v * c2))],
        out_specs=(_pern((c1, vtp)), _pern((v * c2, t)), _pern((c2, 2))),
        scratch_shapes=[pltpu.VMEM((v * c1, t), f32)],
    )(v0, st_v0, col(tg0), col(tb0), m1, bias1, g2)

    v1, st_v1 = pl.pallas_call(
        _tconv_body,
        grid=(n,),
        out_shape=(jax.ShapeDtypeStruct((n, c2, vtp), f32),
                   jax.ShapeDtypeStruct((n, c2, 2), f32)),
        in_specs=[_pern((v * c2, t)), _full((n, c2, 2)), _full((c2, 1)),
                  _full((c2, 1)), _full((c2, NTAP * c2)), _full((c2, 1)),
                  _full((1, vtp))],
        out_specs=(_pern((c2, vtp)), _pern((c2, 2))),
        scratch_shapes=[pltpu.VMEM((c2, vtp + 2 * PAD), f32)],
    )(u1, st_u1, col(sg1), col(sb1), wt1, col(bt1), gap)

    x2, u2, st_u2 = pl.pallas_call(
        _finish_res_sgc_body,
        grid=(n,),
        out_shape=(jax.ShapeDtypeStruct((n, c2, vtp), f32),
                   jax.ShapeDtypeStruct((n, v * c3, t), f32),
                   jax.ShapeDtypeStruct((n, c3, 2), f32)),
        in_specs=[_pern((c2, vtp)), _full((n, c2, 2)), _full((c2, 1)),
                  _full((c2, 1)), _pern((c2, vtp)),
                  _full((v * c3, v * c2)), _full((v * c3, 1)),
                  _full((c3, v * c3))],
        out_specs=(_pern((c2, vtp)), _pern((v * c3, t)), _pern((c3, 2))),
        scratch_shapes=[pltpu.VMEM((v * c2, t), f32)],
    )(v1, st_v1, col(tg1), col(tb1), x1, m2, bias2, g3)

    v2, st_v2, r2, st_r2 = pl.pallas_call(
        _tconv_proj_body,
        grid=(n,),
        out_shape=(jax.ShapeDtypeStruct((n, c3, vtp), f32),
                   jax.ShapeDtypeStruct((n, c3, 2), f32),
                   jax.ShapeDtypeStruct((n, c3, vtp), f32),
                   jax.ShapeDtypeStruct((n, c3, 2), f32)),
        in_specs=[_pern((v * c3, t)), _full((n, c3, 2)), _full((c3, 1)),
                  _full((c3, 1)), _full((c3, NTAP * c3)), _full((c3, 1)),
                  _full((1, vtp)), _pern((c2, vtp)), _full((c3, c2)),
                  _full((c3, 1))],
        out_specs=(_pern((c3, vtp)), _pern((c3, 2)),
                   _pern((c3, vtp)), _pern((c3, 2))),
        scratch_shapes=[pltpu.VMEM((c3, vtp + 2 * PAD), f32)],
    )(u2, st_u2, col(sg2), col(sb2), wt2, col(bt2), gap, x2, Wr2, col(br2))

    out = pl.pallas_call(
        _final_body,
        grid=(n,),
        out_shape=jax.ShapeDtypeStruct((n, c3, vtp), f32),
        in_specs=[_pern((c3, vtp)), _full((n, c3, 2)), _full((c3, 1)),
                  _full((c3, 1)), _pern((c3, vtp)), _full((n, c3, 2)),
                  _full((c3, 1)), _full((c3, 1))],
        out_specs=_pern((c3, vtp)),
    )(v2, st_v2, col(tg2), col(tb2), r2, st_r2, col(rg2), col(rb2))

    return (out.reshape(n, c3, v, tp)[:, :, :, PAD:PAD + t]
            .transpose(0, 1, 3, 2))
